# baseline (speedup 1.0000x reference)
import jax
import jax.numpy as jnp
from jax.experimental import pallas as pl
from jax.experimental.pallas import tpu as pltpu

_EPS = 1e-8
_LANES = 128
_SUB = 8


def _deinterleave(y):
    rows = y.shape[0]
    lanes = jax.lax.broadcasted_iota(jnp.int32, (rows, _LANES), 1)
    idx = jnp.where(lanes < 64, 2 * lanes, 2 * lanes - 127)
    g_lo = jnp.take_along_axis(y[:, :_LANES], idx, axis=1)
    g_hi = jnp.take_along_axis(y[:, _LANES:], idx, axis=1)
    t = jnp.concatenate([g_lo[:, :64], g_hi[:, :64]], axis=1)
    e = jnp.concatenate([g_lo[:, 64:], g_hi[:, 64:]], axis=1)
    return t, e


def _nll_body(y2_ref, rrow_ref, o_ref, acc_ref):
    pid = pl.program_id(0)
    ncells = pl.num_programs(0)

    trow, erow = _deinterleave(y2_ref[...])
    rrow = rrow_ref[...]
    ni = rrow.shape[0]

    mx = jnp.max(rrow, keepdims=True)
    ex = jnp.exp(rrow - mx)

    row0 = pl.multiple_of(pid * _SUB, _SUB)
    t8, e8 = _deinterleave(y2_ref[pl.ds(row0, _SUB), :])
    tT = jnp.transpose(t8)

    cols = []
    for c in range(_SUB):
        tjb = jnp.broadcast_to(tT[:, c : c + 1], (_LANES, _LANES))
        acc = jnp.zeros((_LANES, _LANES), jnp.float32)
        for ci in range(ni):
            m = trow[ci : ci + 1, :] >= tjb
            acc = acc + jnp.where(m, ex[ci : ci + 1, :], 0.0)
        cols.append(jnp.sum(acc, axis=1, keepdims=True))

    rss = jnp.transpose(jnp.concatenate(cols, axis=1))
    log_loss = jnp.log(rss + _EPS) + mx
    r8 = rrow_ref[pl.ds(row0, _SUB), :]
    contrib = (r8 - log_loss) * e8

    @pl.when(pid == 0)
    def _():
        acc_ref[...] = jnp.zeros_like(acc_ref)

    acc_ref[...] += contrib

    @pl.when(pid == ncells - 1)
    def _():
        s_c = jnp.sum(jnp.sum(acc_ref[...], axis=1, keepdims=True), axis=0, keepdims=True)
        s_e = jnp.sum(jnp.sum(erow, axis=1, keepdims=True), axis=0, keepdims=True)
        n = jnp.float32(ni * _LANES)
        o_ref[...] = -s_c / (s_e * n)


def kernel(risk_pred, y_true):
    n = risk_pred.shape[0]
    ni = n // _LANES
    y2 = y_true.reshape(ni, 2 * _LANES)
    rrow = risk_pred.reshape(ni, _LANES)

    nll = pl.pallas_call(
        _nll_body,
        grid=(ni // _SUB,),
        in_specs=[
            pl.BlockSpec((ni, 2 * _LANES), lambda i: (0, 0)),
            pl.BlockSpec((ni, _LANES), lambda i: (0, 0)),
        ],
        out_specs=pl.BlockSpec((1, 1), lambda i: (0, 0)),
        out_shape=jax.ShapeDtypeStruct((1, 1), jnp.float32),
        scratch_shapes=[pltpu.VMEM((_SUB, _LANES), jnp.float32)],
        compiler_params=pltpu.CompilerParams(
            dimension_semantics=("arbitrary",),
        ),
    )(y2, rrow)
    return nll[0, 0]


# speedup vs baseline: 1.0887x; 1.0887x over previous
import jax
import jax.numpy as jnp
from jax.experimental import pallas as pl
from jax.experimental.pallas import tpu as pltpu

_EPS = 1e-8
_LANES = 128
_SUB = 8


def _nll_body(thbm_ref, rhbm_ref, ehbm_ref, o_ref,
              t_ref, r_ref, e_ref, acc_ref, sem_ref):
    pid = pl.program_id(0)
    ncells = pl.num_programs(0)

    @pl.when(pid == 0)
    def _():
        ct = pltpu.make_async_copy(thbm_ref, t_ref, sem_ref.at[0])
        cr = pltpu.make_async_copy(rhbm_ref, r_ref, sem_ref.at[1])
        ce = pltpu.make_async_copy(ehbm_ref, e_ref, sem_ref.at[2])
        ct.start(); cr.start(); ce.start()
        ct.wait(); cr.wait(); ce.wait()
        acc_ref[...] = jnp.zeros_like(acc_ref)

    trow = t_ref[...]
    rrow = r_ref[...]
    ni = rrow.shape[0]

    mx = jnp.max(rrow, keepdims=True)
    ex = jnp.exp(rrow - mx)

    row0 = pl.multiple_of(pid * _SUB, _SUB)
    t8 = t_ref[pl.ds(row0, _SUB), :]
    tT = jnp.transpose(t8)

    cols = []
    for c in range(_SUB):
        tjb = jnp.broadcast_to(tT[:, c : c + 1], (_LANES, _LANES))
        acc = jnp.zeros((_LANES, _LANES), jnp.float32)
        for ci in range(ni):
            m = trow[ci : ci + 1, :] >= tjb
            acc = acc + jnp.where(m, ex[ci : ci + 1, :], 0.0)
        cols.append(jnp.sum(acc, axis=1, keepdims=True))

    rss = jnp.transpose(jnp.concatenate(cols, axis=1))
    log_loss = jnp.log(rss + _EPS) + mx
    r8 = r_ref[pl.ds(row0, _SUB), :]
    e8 = e_ref[pl.ds(row0, _SUB), :]
    acc_ref[...] += (r8 - log_loss) * e8

    @pl.when(pid == ncells - 1)
    def _():
        erow = e_ref[...]
        s_c = jnp.sum(jnp.sum(acc_ref[...], axis=1, keepdims=True), axis=0, keepdims=True)
        s_e = jnp.sum(jnp.sum(erow, axis=1, keepdims=True), axis=0, keepdims=True)
        n = jnp.float32(ni * _LANES)
        o_ref[...] = -s_c / (s_e * n)


def kernel(risk_pred, y_true):
    n = risk_pred.shape[0]
    ni = n // _LANES
    trow = y_true[:, 0].reshape(ni, _LANES)
    erow = y_true[:, 1].reshape(ni, _LANES)
    rrow = risk_pred.reshape(ni, _LANES)

    nll = pl.pallas_call(
        _nll_body,
        grid=(ni // _SUB,),
        in_specs=[
            pl.BlockSpec(memory_space=pl.ANY),
            pl.BlockSpec(memory_space=pl.ANY),
            pl.BlockSpec(memory_space=pl.ANY),
        ],
        out_specs=pl.BlockSpec((1, 1), lambda i: (0, 0)),
        out_shape=jax.ShapeDtypeStruct((1, 1), jnp.float32),
        scratch_shapes=[
            pltpu.VMEM((ni, _LANES), jnp.float32),
            pltpu.VMEM((ni, _LANES), jnp.float32),
            pltpu.VMEM((ni, _LANES), jnp.float32),
            pltpu.VMEM((_SUB, _LANES), jnp.float32),
            pltpu.SemaphoreType.DMA((3,)),
        ],
        compiler_params=pltpu.CompilerParams(
            dimension_semantics=("arbitrary",),
        ),
    )(trow, rrow, erow)
    return nll[0, 0]


# speedup vs baseline: 1.0902x; 1.0014x over previous
import jax
import jax.numpy as jnp
from jax.experimental import pallas as pl
from jax.experimental.pallas import tpu as pltpu

_EPS = 1e-8
_LANES = 128
_SUB = 8


def _nll_body(thbm_ref, rhbm_ref, ehbm_ref, o_ref,
              t_ref, r_ref, e_ref, acc_ref, tT3_ref, ex_ref, mx_ref, sem_ref):
    pid = pl.program_id(0)
    ncells = pl.num_programs(0)

    @pl.when(pid == 0)
    def _():
        ct = pltpu.make_async_copy(thbm_ref, t_ref, sem_ref.at[0])
        cr = pltpu.make_async_copy(rhbm_ref, r_ref, sem_ref.at[1])
        ce = pltpu.make_async_copy(ehbm_ref, e_ref, sem_ref.at[2])
        ct.start(); cr.start(); ce.start()
        ct.wait(); cr.wait(); ce.wait()
        acc_ref[...] = jnp.zeros_like(acc_ref)
        mxv = jnp.max(r_ref[...], keepdims=True)
        mx_ref[...] = mxv
        ex_ref[...] = jnp.exp(r_ref[...] - mxv)
        for g in range(tT3_ref.shape[0]):
            tT3_ref[g] = jnp.transpose(t_ref[g * _SUB : (g + 1) * _SUB, :])

    trow = t_ref[...]
    ni = trow.shape[0]
    mx = mx_ref[...]
    ex = ex_ref[...]

    row0 = pl.multiple_of(pid * _SUB, _SUB)
    tT = tT3_ref[pid]

    cols = []
    for c in range(_SUB):
        tjb = jnp.broadcast_to(tT[:, c : c + 1], (_LANES, _LANES))
        acc = jnp.zeros((_LANES, _LANES), jnp.float32)
        for ci in range(ni):
            m = trow[ci : ci + 1, :] >= tjb
            acc = acc + jnp.where(m, ex[ci : ci + 1, :], 0.0)
        cols.append(jnp.sum(acc, axis=1, keepdims=True))

    rss = jnp.transpose(jnp.concatenate(cols, axis=1))
    log_loss = jnp.log(rss + _EPS) + mx
    r8 = r_ref[pl.ds(row0, _SUB), :]
    e8 = e_ref[pl.ds(row0, _SUB), :]
    acc_ref[...] += (r8 - log_loss) * e8

    @pl.when(pid == ncells - 1)
    def _():
        erow = e_ref[...]
        s_c = jnp.sum(jnp.sum(acc_ref[...], axis=1, keepdims=True), axis=0, keepdims=True)
        s_e = jnp.sum(jnp.sum(erow, axis=1, keepdims=True), axis=0, keepdims=True)
        n = jnp.float32(ni * _LANES)
        o_ref[...] = -s_c / (s_e * n)


def kernel(risk_pred, y_true):
    n = risk_pred.shape[0]
    ni = n // _LANES
    trow = y_true[:, 0].reshape(ni, _LANES)
    erow = y_true[:, 1].reshape(ni, _LANES)
    rrow = risk_pred.reshape(ni, _LANES)

    nll = pl.pallas_call(
        _nll_body,
        grid=(ni // _SUB,),
        in_specs=[
            pl.BlockSpec(memory_space=pl.ANY),
            pl.BlockSpec(memory_space=pl.ANY),
            pl.BlockSpec(memory_space=pl.ANY),
        ],
        out_specs=pl.BlockSpec((1, 1), lambda i: (0, 0)),
        out_shape=jax.ShapeDtypeStruct((1, 1), jnp.float32),
        scratch_shapes=[
            pltpu.VMEM((ni, _LANES), jnp.float32),
            pltpu.VMEM((ni, _LANES), jnp.float32),
            pltpu.VMEM((ni, _LANES), jnp.float32),
            pltpu.VMEM((_SUB, _LANES), jnp.float32),
            pltpu.VMEM((ni // _SUB, _LANES, _SUB), jnp.float32),
            pltpu.VMEM((ni, _LANES), jnp.float32),
            pltpu.VMEM((1, 1), jnp.float32),
            pltpu.SemaphoreType.DMA((3,)),
        ],
        compiler_params=pltpu.CompilerParams(
            dimension_semantics=("arbitrary",),
        ),
    )(trow, rrow, erow)
    return nll[0, 0]
